# Initial kernel scaffold; baseline (speedup 1.0000x reference)
import jax
import jax.numpy as jnp
from jax.experimental import pallas as pl


def kernel(x, edge_index, W, b):
    raise NotImplementedError("write your pallas kernel here")


# speedup vs baseline: 3.1323x; 3.1323x over previous
import functools

import jax
import jax.numpy as jnp
from jax import lax
from jax.experimental import pallas as pl
from jax.experimental.pallas import tpu as pltpu
from jax.experimental.pallas import tpu_sc as plsc

N_NODES = 10000
N_EDGES = 320000
D_FEAT = 128

NC = 2
NS = 16
NW = NC * NS
BLK = 128

BLOCKS_PER_TILE = 80
E_PAD = NW * BLK * BLOCKS_PER_TILE

ACC_ROWS = 10240
STRIPE = ACC_ROWS // NS


def _sc_pool_body(x_hbm, src_hbm, dst_hbm, z_hbm, out_hbm,
                  sidx, didx, rows, acc, sem):
    c = lax.axis_index("c")
    s = lax.axis_index("s")
    wid = s * NC + c

    pltpu.sync_copy(z_hbm, acc.at[pl.ds(s * STRIPE, STRIPE)])

    row0 = wid * BLOCKS_PER_TILE
    pltpu.sync_copy(src_hbm.at[pl.ds(row0, BLOCKS_PER_TILE)], sidx)
    pltpu.sync_copy(dst_hbm.at[pl.ds(row0, BLOCKS_PER_TILE)], didx)

    plsc.subcore_barrier()

    @pl.loop(0, BLOCKS_PER_TILE)
    def _(j):
        pltpu.async_copy(x_hbm.at[sidx.at[j]], rows, sem).wait()
        pltpu.sync_copy(rows, acc.at[didx.at[j]], add=True)

    plsc.subcore_barrier()

    pltpu.sync_copy(acc.at[pl.ds(s * STRIPE, STRIPE)],
                    out_hbm.at[c, pl.ds(s * STRIPE, STRIPE)])


_sc_pool = pl.kernel(
    _sc_pool_body,
    out_type=jax.ShapeDtypeStruct((NC, ACC_ROWS, D_FEAT), jnp.float32),
    mesh=plsc.VectorSubcoreMesh(core_axis_name="c", subcore_axis_name="s"),
    scratch_types=[
        pltpu.VMEM((BLOCKS_PER_TILE, BLK), jnp.int32),
        pltpu.VMEM((BLOCKS_PER_TILE, BLK), jnp.int32),
        pltpu.VMEM((BLK, D_FEAT), jnp.float32),
        pltpu.VMEM_SHARED((ACC_ROWS, D_FEAT), jnp.float32),
        pltpu.SemaphoreType.DMA,
    ],
)


ROW_BLK = 1000


def _tc_dense_body(x_ref, p_ref, w1_ref, w2_ref, b_ref, o_ref):
    pooled = p_ref[0] + p_ref[1]
    h = jnp.dot(x_ref[...], w1_ref[...], preferred_element_type=jnp.float32)
    h = h + jnp.dot(pooled, w2_ref[...], preferred_element_type=jnp.float32)
    o_ref[...] = jnp.maximum(h + b_ref[...], 0.0)


def _tc_dense(x, partials, w1, w2, b2d):
    n = x.shape[0]
    grid = n // ROW_BLK
    return pl.pallas_call(
        _tc_dense_body,
        grid=(grid,),
        in_specs=[
            pl.BlockSpec((ROW_BLK, D_FEAT), lambda i: (i, 0)),
            pl.BlockSpec((NC, ROW_BLK, D_FEAT), lambda i: (0, i, 0)),
            pl.BlockSpec((D_FEAT, D_FEAT), lambda i: (0, 0)),
            pl.BlockSpec((D_FEAT, D_FEAT), lambda i: (0, 0)),
            pl.BlockSpec((1, D_FEAT), lambda i: (0, 0)),
        ],
        out_specs=pl.BlockSpec((ROW_BLK, D_FEAT), lambda i: (i, 0)),
        out_shape=jax.ShapeDtypeStruct((n, D_FEAT), jnp.float32),
    )(x, partials, w1, w2, b2d)


def kernel(x, edge_index, W, b):
    src = edge_index[0].astype(jnp.int32)
    dst = edge_index[1].astype(jnp.int32)
    pad = E_PAD - N_EDGES
    src = jnp.concatenate([src, jnp.zeros((pad,), jnp.int32)])
    dst = jnp.concatenate([dst, jnp.full((pad,), N_NODES, jnp.int32)])
    src2 = src.reshape(E_PAD // BLK, BLK)
    dst2 = dst.reshape(E_PAD // BLK, BLK)
    zrows = jnp.zeros((STRIPE, D_FEAT), jnp.float32)

    partials = _sc_pool(x, src2, dst2, zrows)[:, :N_NODES]

    w1 = W[:D_FEAT]
    w2 = W[D_FEAT:]
    return _tc_dense(x, partials, w1, w2, b.reshape(1, D_FEAT))


# speedup vs baseline: 3.4134x; 1.0897x over previous
import functools

import jax
import jax.numpy as jnp
from jax import lax
from jax.experimental import pallas as pl
from jax.experimental.pallas import tpu as pltpu
from jax.experimental.pallas import tpu_sc as plsc

N_NODES = 10000
N_EDGES = 320000
D_FEAT = 128

NC = 2
NS = 16
NW = NC * NS
BLK = 128

BLOCKS_PER_TILE = 80
E_PAD = NW * BLK * BLOCKS_PER_TILE

ACC_ROWS = 10112
STRIPE = ACC_ROWS // NS
CHUNK = 16


def _sc_pool_body(x_hbm, src_hbm, dst_hbm, z_hbm, out_hbm,
                  sidx, didx, rows, acc, sem_a, sem_b):
    c = lax.axis_index("c")
    s = lax.axis_index("s")
    wid = s * NC + c

    pltpu.sync_copy(z_hbm, acc.at[pl.ds(s * STRIPE, STRIPE)])

    row0 = wid * BLOCKS_PER_TILE

    plsc.subcore_barrier()

    def fire(j, buf, sem):
        pltpu.async_copy(x_hbm.at[sidx.at[j]], rows.at[buf], sem)

    def drain_and_scatter(j, buf, sem):
        pltpu.make_async_copy(x_hbm.at[sidx.at[j]], rows.at[buf], sem).wait()
        pltpu.sync_copy(rows.at[buf], acc.at[didx.at[j]], add=True)

    @pl.loop(0, BLOCKS_PER_TILE // CHUNK)
    def _(ci):
        base = row0 + ci * CHUNK
        pltpu.sync_copy(src_hbm.at[pl.ds(base, CHUNK)], sidx)
        pltpu.sync_copy(dst_hbm.at[pl.ds(base, CHUNK)], didx)

        fire(0, 0, sem_a)

        @pl.loop(0, CHUNK, step=2)
        def _(j):
            fire(j + 1, 1, sem_b)
            drain_and_scatter(j, 0, sem_a)

            @pl.when(j + 2 < CHUNK)
            def _():
                fire(j + 2, 0, sem_a)

            drain_and_scatter(j + 1, 1, sem_b)

    plsc.subcore_barrier()

    pltpu.sync_copy(acc.at[pl.ds(s * STRIPE, STRIPE)],
                    out_hbm.at[c, pl.ds(s * STRIPE, STRIPE)])


_sc_pool = pl.kernel(
    _sc_pool_body,
    out_type=jax.ShapeDtypeStruct((NC, ACC_ROWS, D_FEAT), jnp.float32),
    mesh=plsc.VectorSubcoreMesh(core_axis_name="c", subcore_axis_name="s"),
    scratch_types=[
        pltpu.VMEM((CHUNK, BLK), jnp.int32),
        pltpu.VMEM((CHUNK, BLK), jnp.int32),
        pltpu.VMEM((2, BLK, D_FEAT), jnp.float32),
        pltpu.VMEM_SHARED((ACC_ROWS, D_FEAT), jnp.float32),
        pltpu.SemaphoreType.DMA,
        pltpu.SemaphoreType.DMA,
    ],
)


ROW_BLK = 1000


def _tc_dense_body(x_ref, p_ref, w1_ref, w2_ref, b_ref, o_ref):
    pooled = p_ref[0] + p_ref[1]
    h = jnp.dot(x_ref[...], w1_ref[...], preferred_element_type=jnp.float32)
    h = h + jnp.dot(pooled, w2_ref[...], preferred_element_type=jnp.float32)
    o_ref[...] = jnp.maximum(h + b_ref[...], 0.0)


def _tc_dense(x, partials, w1, w2, b2d):
    n = x.shape[0]
    grid = n // ROW_BLK
    return pl.pallas_call(
        _tc_dense_body,
        grid=(grid,),
        in_specs=[
            pl.BlockSpec((ROW_BLK, D_FEAT), lambda i: (i, 0)),
            pl.BlockSpec((NC, ROW_BLK, D_FEAT), lambda i: (0, i, 0)),
            pl.BlockSpec((D_FEAT, D_FEAT), lambda i: (0, 0)),
            pl.BlockSpec((D_FEAT, D_FEAT), lambda i: (0, 0)),
            pl.BlockSpec((1, D_FEAT), lambda i: (0, 0)),
        ],
        out_specs=pl.BlockSpec((ROW_BLK, D_FEAT), lambda i: (i, 0)),
        out_shape=jax.ShapeDtypeStruct((n, D_FEAT), jnp.float32),
    )(x, partials, w1, w2, b2d)


def kernel(x, edge_index, W, b):
    src = edge_index[0].astype(jnp.int32)
    dst = edge_index[1].astype(jnp.int32)
    pad = E_PAD - N_EDGES
    src = jnp.concatenate([src, jnp.zeros((pad,), jnp.int32)])
    dst = jnp.concatenate([dst, jnp.full((pad,), N_NODES, jnp.int32)])
    src2 = src.reshape(E_PAD // BLK, BLK)
    dst2 = dst.reshape(E_PAD // BLK, BLK)
    zrows = jnp.zeros((STRIPE, D_FEAT), jnp.float32)

    partials = _sc_pool(x, src2, dst2, zrows)[:, :N_NODES]

    w1 = W[:D_FEAT]
    w2 = W[D_FEAT:]
    return _tc_dense(x, partials, w1, w2, b.reshape(1, D_FEAT))
